# Initial kernel scaffold; baseline (speedup 1.0000x reference)
import jax
import jax.numpy as jnp
from jax.experimental import pallas as pl


def kernel(input0, input1, table):
    raise NotImplementedError("write your pallas kernel here")


# speedup vs baseline: 3.3269x; 3.3269x over previous
import functools

import jax
import jax.numpy as jnp
from jax import lax
from jax.experimental import pallas as pl
from jax.experimental.pallas import tpu as pltpu
from jax.experimental.pallas import tpu_sc as plsc

B = 4096
J = 100
D = 10
V = 100
N = B * J
NW = 32
PER_W = N // NW
CHUNK_ROWS = 3200
N_CHUNKS = PER_W // CHUNK_ROWS
CHUNK_WORDS = CHUNK_ROWS * D
PERIODS = CHUNK_ROWS // 8
LANE = 16


def _gather_body(idx_hbm, table_hbm, out_hbm, idx_v, table_v, out_v, sem):
    wid = lax.axis_index("s") * 2 + lax.axis_index("c")
    pltpu.sync_copy(table_hbm, table_v)
    pltpu.sync_copy(idx_hbm.at[pl.ds(wid * PER_W, PER_W)], idx_v)

    lane = lax.broadcasted_iota(jnp.int32, (LANE,), 0)
    row_off = [(p * LANE + lane) // D for p in range(5)]
    col = [(p * LANE + lane) % D for p in range(5)]

    for c in range(N_CHUNKS):
        row_base = c * CHUNK_ROWS

        def period(g, carry):
            base16 = jnp.full((LANE,), row_base + g * 8, jnp.int32)
            for p in range(5):
                rows16 = plsc.load_gather(idx_v, [base16 + row_off[p]])
                words16 = rows16 * D + col[p]
                vals = plsc.load_gather(table_v, [words16])
                out_v[pl.ds(g * 80 + p * LANE, LANE)] = vals
            return carry

        lax.fori_loop(0, PERIODS, period, 0)
        pltpu.sync_copy(
            out_v, out_hbm.at[pl.ds(wid * PER_W * D + c * CHUNK_WORDS, CHUNK_WORDS)]
        )


def _make_gather():
    mesh = plsc.VectorSubcoreMesh(core_axis_name="c", subcore_axis_name="s")
    return functools.partial(
        pl.kernel,
        mesh=mesh,
        compiler_params=pltpu.CompilerParams(needs_layout_passes=False),
        out_type=jax.ShapeDtypeStruct((N * D,), jnp.float32),
        scratch_types=[
            pltpu.VMEM((PER_W,), jnp.int32),
            pltpu.VMEM((V * D,), jnp.float32),
            pltpu.VMEM((CHUNK_WORDS,), jnp.float32),
            pltpu.SemaphoreType.DMA,
        ],
    )(_gather_body)


_gather = _make_gather()


def _scale_body(x_ref, o_ref):
    o_ref[...] = x_ref[...] * 0.5 + 2.0


def kernel(input0, input1, table):
    output0 = pl.pallas_call(
        _scale_body,
        out_shape=jax.ShapeDtypeStruct(input0.shape, input0.dtype),
    )(input0)
    idx = input1.reshape(N).astype(jnp.int32)
    out1 = _gather(idx, table.reshape(V * D))
    return (output0, out1.reshape(B, J, D))


# speedup vs baseline: 29.8493x; 8.9721x over previous
import functools

import jax
import jax.numpy as jnp
from jax import lax
from jax.experimental import pallas as pl
from jax.experimental.pallas import tpu as pltpu
from jax.experimental.pallas import tpu_sc as plsc

B = 4096
J = 100
D = 10
V = 100
NW = 32
LANE = 16
JTF = 12
FULL_UNITS = D * JTF
VPB = B // LANE


def _make_body():
    def body(idx_hbm, table_hbm, out_hbm, table_v, idx_v, out_a, out_b, sem):
        wid = lax.axis_index("s") * 2 + lax.axis_index("c")
        pltpu.sync_copy(table_hbm, table_v)
        bufs = (out_a, out_b)

        def unit_compute(d, buf, nrows):
            @plsc.parallel_loop(0, VPB, 1, unroll=1)
            def vloop(i):
                for jl in range(nrows):
                    idx16 = idx_v[jl, pl.ds(i * LANE, LANE)]
                    vals = plsc.load_gather(table_v, [idx16 + d * V])
                    buf[jl, pl.ds(i * LANE, LANE)] = vals

        def run_full(u, buf, sync):
            d = u // JTF
            j0 = pl.multiple_of((u % JTF) * 8, 8)
            pltpu.sync_copy(idx_hbm.at[pl.ds(j0, 8), :], idx_v)
            unit_compute(d, buf, 8)
            if sync:
                pltpu.sync_copy(buf, out_hbm.at[d, pl.ds(j0, 8), :])
                return None
            return pltpu.async_copy(buf, out_hbm.at[d, pl.ds(j0, 8), :], sem)

        h0 = run_full(wid, bufs[0], False)
        h1 = run_full(NW + wid, bufs[1], False)
        h0.wait()
        h2 = run_full(2 * NW + wid, bufs[0], False)
        h1.wait()

        @pl.when(wid < FULL_UNITS - 3 * NW)
        def _k3():
            run_full(3 * NW + wid, bufs[1], True)

        h2.wait()

        @pl.when(jnp.logical_or(wid >= 24, wid < 2))
        def _partial():
            pd = jnp.where(wid < 2, wid + 8, wid - 24)
            pltpu.sync_copy(idx_hbm.at[pl.ds(96, 4), :], idx_v.at[pl.ds(0, 4), :])
            unit_compute(pd, bufs[0], 4)
            pltpu.sync_copy(bufs[0].at[pl.ds(0, 4), :], out_hbm.at[pd, pl.ds(96, 4), :])

    return body


def _make_gather():
    mesh = plsc.VectorSubcoreMesh(core_axis_name="c", subcore_axis_name="s")
    return functools.partial(
        pl.kernel,
        mesh=mesh,
        compiler_params=pltpu.CompilerParams(needs_layout_passes=False),
        out_type=jax.ShapeDtypeStruct((D, J, B), jnp.float32),
        scratch_types=[
            pltpu.VMEM((D * V,), jnp.float32),
            pltpu.VMEM((8, B), jnp.int32),
            pltpu.VMEM((8, B), jnp.float32),
            pltpu.VMEM((8, B), jnp.float32),
            pltpu.SemaphoreType.DMA,
        ],
    )(_make_body())


_gather = _make_gather()


def _scale_body(x_ref, o_ref):
    o_ref[...] = x_ref[...] * 0.5 + 2.0


def kernel(input0, input1, table):
    output0 = pl.pallas_call(
        _scale_body,
        out_shape=jax.ShapeDtypeStruct(input0.shape, input0.dtype),
    )(input0)
    idx_t = input1.astype(jnp.int32).T
    table_t = table.T.reshape(D * V)
    out_t = _gather(idx_t, table_t)
    return (output0, out_t.transpose(2, 1, 0))
